# baseline (speedup 1.0000x reference)
import functools

import jax
import jax.numpy as jnp
from jax.experimental import pallas as pl

_EMB = 64
_NUM_LAYERS = 2
_BLOCK = 1024


def _dense_block_kernel(x_ref, w_ref, b_ref, o_ref, *, relu):
    acc = jnp.dot(x_ref[...], w_ref[...], preferred_element_type=jnp.float32)
    acc = acc + b_ref[...][None, :]
    if relu:
        acc = jnp.maximum(acc, 0.0)
    o_ref[...] = acc


def _dense(x, w, b, relu):
    n, k = x.shape
    kout = w.shape[1]
    n_pad = ((n + _BLOCK - 1) // _BLOCK) * _BLOCK
    if n_pad != n:
        x = jnp.pad(x, ((0, n_pad - n), (0, 0)))
    grid = (n_pad // _BLOCK,)
    out = pl.pallas_call(
        functools.partial(_dense_block_kernel, relu=relu),
        grid=grid,
        in_specs=[
            pl.BlockSpec((_BLOCK, k), lambda i: (i, 0)),
            pl.BlockSpec((k, kout), lambda i: (0, 0)),
            pl.BlockSpec((kout,), lambda i: (0,)),
        ],
        out_specs=pl.BlockSpec((_BLOCK, kout), lambda i: (i, 0)),
        out_shape=jax.ShapeDtypeStruct((n_pad, kout), jnp.float32),
    )(x, w, b)
    if n_pad != n:
        out = out[:n]
    return out


def kernel(pos, x, x1, x2, x3, batch_idx, adj_0_0_1, adj_0_0_2, adj_1_0,
           adj_2_0, adj_0_2, adj_2_3, adj_3_2, W_emb0, b_emb0, W_emb1,
           b_emb1, W_emb2, b_emb2, W_emb3, b_emb3, W_msg, b_msg, W_upd,
           b_upd, W_out, b_out):
    h = [
        _dense(x, W_emb0, b_emb0, relu=False),
        _dense(x1, W_emb1, b_emb1, relu=False),
        _dense(x2, W_emb2, b_emb2, relu=False),
        _dense(x3, W_emb3, b_emb3, relu=False),
    ]
    adjs = [
        (adj_0_0_1, 0, 0, True),
        (adj_0_0_2, 0, 0, True),
        (adj_1_0, 1, 0, False),
        (adj_2_0, 2, 0, False),
        (adj_0_2, 0, 2, False),
        (adj_2_3, 2, 3, False),
        (adj_3_2, 3, 2, False),
    ]
    for l in range(_NUM_LAYERS):
        agg = [jnp.zeros_like(hr) for hr in h]
        for a, (adj, rr, sr, use_dist) in enumerate(adjs):
            dst, src = adj[0], adj[1]
            if use_dist:
                d = pos[dst] - pos[src]
                inv = jnp.sqrt(jnp.sum(d * d, axis=-1, keepdims=True) + 1e-12)
            else:
                inv = jnp.zeros((adj.shape[1], 1), dtype=jnp.float32)
            m_in = jnp.concatenate([h[rr][dst], h[sr][src], inv], axis=-1)
            m = _dense(m_in, W_msg[l, a], b_msg[l, a], relu=True)
            agg[rr] = agg[rr] + jax.ops.segment_sum(
                m, dst, num_segments=h[rr].shape[0])
        new_h = []
        for r in range(4):
            u_in = jnp.concatenate([h[r], agg[r]], axis=-1)
            new_h.append(h[r] + _dense(u_in, W_upd[l, r], b_upd[l, r],
                                       relu=True))
        h = new_h
    n3 = x3.shape[0]
    sums = jax.ops.segment_sum(h[0], batch_idx, num_segments=n3)
    cnt = jax.ops.segment_sum(
        jnp.ones((h[0].shape[0], 1), dtype=jnp.float32), batch_idx,
        num_segments=n3)
    pooled = sums / jnp.maximum(cnt, 1.0)
    return _dense(pooled, W_out, b_out, relu=False)


# speedup vs baseline: 1.0846x; 1.0846x over previous
import functools

import jax
import jax.numpy as jnp
from jax.experimental import pallas as pl

_EMB = 64
_NUM_LAYERS = 2
_BLOCK = 2048


def _pad_rows(x, n_pad):
    n = x.shape[0]
    if n_pad != n:
        x = jnp.pad(x, ((0, n_pad - n), (0, 0)))
    return x


def _dense_block_kernel(x_ref, w_ref, b_ref, o_ref, *, relu):
    acc = jnp.dot(x_ref[...], w_ref[...], preferred_element_type=jnp.float32)
    acc = acc + b_ref[...][None, :]
    if relu:
        acc = jnp.maximum(acc, 0.0)
    o_ref[...] = acc


def _dense(x, w, b, relu=False, block=_BLOCK):
    n, k = x.shape
    kout = w.shape[1]
    n_pad = ((n + block - 1) // block) * block
    x = _pad_rows(x, n_pad)
    out = pl.pallas_call(
        functools.partial(_dense_block_kernel, relu=relu),
        grid=(n_pad // block,),
        in_specs=[
            pl.BlockSpec((block, k), lambda i: (i, 0)),
            pl.BlockSpec((k, kout), lambda i: (0, 0)),
            pl.BlockSpec((kout,), lambda i: (0,)),
        ],
        out_specs=pl.BlockSpec((block, kout), lambda i: (i, 0)),
        out_shape=jax.ShapeDtypeStruct((n_pad, kout), jnp.float32),
    )(x, w, b)
    return out[:n] if n_pad != n else out


def _msg_block_kernel(hd_ref, hs_ref, inv_ref, wd_ref, ws_ref, wi_ref,
                      b_ref, o_ref):
    acc = jnp.dot(hd_ref[...], wd_ref[...], preferred_element_type=jnp.float32)
    acc += jnp.dot(hs_ref[...], ws_ref[...], preferred_element_type=jnp.float32)
    acc += inv_ref[...] * wi_ref[...]
    acc += b_ref[...][None, :]
    o_ref[...] = jnp.maximum(acc, 0.0)


def _message(hd, hs, inv, w, b):
    n = hd.shape[0]
    n_pad = ((n + _BLOCK - 1) // _BLOCK) * _BLOCK
    hd = _pad_rows(hd, n_pad)
    hs = _pad_rows(hs, n_pad)
    inv = _pad_rows(inv, n_pad)
    wd, ws, wi = w[:_EMB], w[_EMB:2 * _EMB], w[2 * _EMB:]
    out = pl.pallas_call(
        _msg_block_kernel,
        grid=(n_pad // _BLOCK,),
        in_specs=[
            pl.BlockSpec((_BLOCK, _EMB), lambda i: (i, 0)),
            pl.BlockSpec((_BLOCK, _EMB), lambda i: (i, 0)),
            pl.BlockSpec((_BLOCK, 1), lambda i: (i, 0)),
            pl.BlockSpec((_EMB, _EMB), lambda i: (0, 0)),
            pl.BlockSpec((_EMB, _EMB), lambda i: (0, 0)),
            pl.BlockSpec((1, _EMB), lambda i: (0, 0)),
            pl.BlockSpec((_EMB,), lambda i: (0,)),
        ],
        out_specs=pl.BlockSpec((_BLOCK, _EMB), lambda i: (i, 0)),
        out_shape=jax.ShapeDtypeStruct((n_pad, _EMB), jnp.float32),
    )(hd, hs, inv, wd, ws, wi, b)
    return out[:n] if n_pad != n else out


def _upd_block_kernel(h_ref, a_ref, wh_ref, wa_ref, b_ref, o_ref):
    acc = jnp.dot(h_ref[...], wh_ref[...], preferred_element_type=jnp.float32)
    acc += jnp.dot(a_ref[...], wa_ref[...], preferred_element_type=jnp.float32)
    acc += b_ref[...][None, :]
    o_ref[...] = h_ref[...] + jnp.maximum(acc, 0.0)


def _update(h, agg, w, b):
    n = h.shape[0]
    n_pad = ((n + _BLOCK - 1) // _BLOCK) * _BLOCK
    h = _pad_rows(h, n_pad)
    agg = _pad_rows(agg, n_pad)
    wh, wa = w[:_EMB], w[_EMB:]
    out = pl.pallas_call(
        _upd_block_kernel,
        grid=(n_pad // _BLOCK,),
        in_specs=[
            pl.BlockSpec((_BLOCK, _EMB), lambda i: (i, 0)),
            pl.BlockSpec((_BLOCK, _EMB), lambda i: (i, 0)),
            pl.BlockSpec((_EMB, _EMB), lambda i: (0, 0)),
            pl.BlockSpec((_EMB, _EMB), lambda i: (0, 0)),
            pl.BlockSpec((_EMB,), lambda i: (0,)),
        ],
        out_specs=pl.BlockSpec((_BLOCK, _EMB), lambda i: (i, 0)),
        out_shape=jax.ShapeDtypeStruct((n_pad, _EMB), jnp.float32),
    )(h, agg, wh, wa, b)
    return out[:n] if n_pad != n else out


def kernel(pos, x, x1, x2, x3, batch_idx, adj_0_0_1, adj_0_0_2, adj_1_0,
           adj_2_0, adj_0_2, adj_2_3, adj_3_2, W_emb0, b_emb0, W_emb1,
           b_emb1, W_emb2, b_emb2, W_emb3, b_emb3, W_msg, b_msg, W_upd,
           b_upd, W_out, b_out):
    h = [
        _dense(x, W_emb0, b_emb0),
        _dense(x1, W_emb1, b_emb1),
        _dense(x2, W_emb2, b_emb2),
        _dense(x3, W_emb3, b_emb3),
    ]
    adjs = [
        (adj_0_0_1, 0, 0, True),
        (adj_0_0_2, 0, 0, True),
        (adj_1_0, 1, 0, False),
        (adj_2_0, 2, 0, False),
        (adj_0_2, 0, 2, False),
        (adj_2_3, 2, 3, False),
        (adj_3_2, 3, 2, False),
    ]
    for l in range(_NUM_LAYERS):
        agg = [jnp.zeros_like(hr) for hr in h]
        for a, (adj, rr, sr, use_dist) in enumerate(adjs):
            dst, src = adj[0], adj[1]
            if use_dist:
                d = pos[dst] - pos[src]
                inv = jnp.sqrt(jnp.sum(d * d, axis=-1, keepdims=True) + 1e-12)
            else:
                inv = jnp.zeros((adj.shape[1], 1), dtype=jnp.float32)
            m = _message(h[rr][dst], h[sr][src], inv, W_msg[l, a], b_msg[l, a])
            agg[rr] = agg[rr] + jax.ops.segment_sum(
                m, dst, num_segments=h[rr].shape[0])
        h = [_update(h[r], agg[r], W_upd[l, r], b_upd[l, r]) for r in range(4)]
    n3 = x3.shape[0]
    sums = jax.ops.segment_sum(h[0], batch_idx, num_segments=n3)
    cnt = jax.ops.segment_sum(
        jnp.ones((h[0].shape[0], 1), dtype=jnp.float32), batch_idx,
        num_segments=n3)
    pooled = sums / jnp.maximum(cnt, 1.0)
    return _dense(pooled, W_out, b_out)
